# baseline (speedup 1.0000x reference)
import jax
import jax.numpy as jnp
from jax import lax
from jax.experimental import pallas as pl
from jax.experimental.pallas import tpu as pltpu

S, D, H, E, TOPK, F = 2048, 1024, 64, 16, 2, 64

_TT = (((0,), (0,)), ((), ()))


def _moe_body(x_ref, wout_ref, win_ref, wlin_ref, we_hbm, o_ref,
              ws_ref, sem0, sem1):
    x = x_ref[...]
    xb = x.astype(jnp.bfloat16)
    vcol = lax.dot_general(xb, wout_ref[...].astype(jnp.bfloat16), _TT,
                           preferred_element_type=jnp.float32)

    t = jnp.sum(win_ref[...] * vcol, axis=0, keepdims=True)
    tcol = lax.transpose(t, (1, 0))
    g = jnp.sum(wlin_ref[...] * tcol, axis=0, keepdims=True)

    iota = lax.broadcasted_iota(jnp.int32, (1, E), 1)
    m1v = jnp.max(g, axis=1, keepdims=True)
    i1 = jnp.min(jnp.where(g == m1v, iota, E))
    g2 = jnp.where(iota == i1, -jnp.inf, g)
    m2v = jnp.max(g2, axis=1, keepdims=True)
    i2 = jnp.min(jnp.where(g2 == m2v, iota, E))

    cp0 = pltpu.make_async_copy(we_hbm.at[pl.ds(i1, 1)],
                                ws_ref.at[pl.ds(0, 1)], sem0)
    cp1 = pltpu.make_async_copy(we_hbm.at[pl.ds(i2, 1)],
                                ws_ref.at[pl.ds(1, 1)], sem1)
    cp0.start()
    cp1.start()

    w1 = 1.0 / (1.0 + jnp.exp(m2v[0, 0] - m1v[0, 0]))
    w2 = 1.0 - w1

    cp0.wait()
    cp1.wait()

    Wc = jnp.concatenate([ws_ref[0], ws_ref[1]],
                         axis=1).astype(jnp.bfloat16)
    z = jnp.dot(xb, Wc, preferred_element_type=jnp.float32)

    def norm_gelu(zk, wk):
        n = jnp.maximum(
            jnp.sqrt(jnp.sum(zk * zk, axis=-1, keepdims=True)), 1e-12)
        zn = zk / n
        c = jnp.float32(0.7071067811865476)
        return wk * (0.5 * zn * (1.0 + lax.erf(zn * c)))

    o_ref[...] = norm_gelu(z[:, :F], w1) + norm_gelu(z[:, F:], w2)


def kernel(x, W_gate_in, W_gate_lin, W_gate_out, W_experts):
    return pl.pallas_call(
        _moe_body,
        in_specs=[
            pl.BlockSpec((S, D), lambda: (0, 0)),
            pl.BlockSpec((S, 1), lambda: (0, 0)),
            pl.BlockSpec((D, H), lambda: (0, 0)),
            pl.BlockSpec((H, E), lambda: (0, 0)),
            pl.BlockSpec(memory_space=pl.ANY),
        ],
        out_specs=pl.BlockSpec((S, F), lambda: (0, 0)),
        out_shape=jax.ShapeDtypeStruct((S, F), jnp.float32),
        scratch_shapes=[
            pltpu.VMEM((TOPK, D, F), jnp.float32),
            pltpu.SemaphoreType.DMA,
            pltpu.SemaphoreType.DMA,
        ],
    )(x, W_gate_out, W_gate_in, W_gate_lin, W_experts)
